# speedup vs baseline: 5.9569x; 5.9569x over previous
import functools

import jax
import jax.numpy as jnp
from jax import lax
from jax.experimental import pallas as pl
from jax.experimental.pallas import tpu as pltpu
from jax.experimental.pallas import tpu_sc as plsc

N = 10000
E = 320000
P = 200000
D = 128
H = 128
EPS = 1e-5

NC = 2
NS = 16
NW = NC * NS
CH = 128

NPAD = 10240
TJ = -(-E // (NW * CH))
EPADLEN = NW * CH * TJ
PJ = -(-P // (NW * CH))
PPADLEN = NW * CH * PJ
ZROWS = NPAD // NS

_sc_mesh = plsc.VectorSubcoreMesh(core_axis_name="c", subcore_axis_name="s")


@functools.partial(
    pl.kernel, mesh=_sc_mesh,
    out_type=jax.ShapeDtypeStruct((NC, NPAD, 16), jnp.float32),
    scratch_types=[
        pltpu.VMEM((TJ, CH), jnp.int32),
        pltpu.VMEM((CH, 16), jnp.float32),
        pltpu.VMEM_SHARED((NPAD, 16), jnp.float32),
    ],
)
def _deg_kernel(dst_hbm, zeros_hbm, ones_hbm, out_hbm, idx_v, ones_v, shared):
    c = lax.axis_index("c")
    s = lax.axis_index("s")
    wid = s * NC + c
    pltpu.sync_copy(zeros_hbm.at[pl.ds(s * ZROWS, ZROWS)],
                    shared.at[pl.ds(s * ZROWS, ZROWS)])
    pltpu.sync_copy(ones_hbm, ones_v)
    pltpu.sync_copy(dst_hbm.at[wid], idx_v)
    plsc.subcore_barrier()

    def body(j, carry):
        pltpu.sync_copy(ones_v, shared.at[idx_v.at[j]], add=True)
        return carry

    lax.fori_loop(0, TJ, body, 0)
    plsc.subcore_barrier()
    pltpu.sync_copy(shared.at[pl.ds(s * ZROWS, ZROWS)],
                    out_hbm.at[c, pl.ds(s * ZROWS, ZROWS)])


@functools.partial(
    pl.kernel, mesh=_sc_mesh,
    out_type=jax.ShapeDtypeStruct((NC, NPAD, D), jnp.float32),
    scratch_types=[
        pltpu.VMEM((TJ, CH), jnp.int32),
        pltpu.VMEM((TJ, CH), jnp.int32),
        pltpu.VMEM((CH, D), jnp.float32),
        pltpu.VMEM_SHARED((NPAD, D), jnp.float32),
        pltpu.SemaphoreType.DMA,
    ],
)
def _agg_kernel(y_hbm, src_hbm, dst_hbm, zeros_hbm, out_hbm,
                srcv, dstv, rows_v, shared, sem):
    c = lax.axis_index("c")
    s = lax.axis_index("s")
    wid = s * NC + c
    pltpu.sync_copy(zeros_hbm.at[pl.ds(s * ZROWS, ZROWS)],
                    shared.at[pl.ds(s * ZROWS, ZROWS)])
    pltpu.sync_copy(src_hbm.at[wid], srcv)
    pltpu.sync_copy(dst_hbm.at[wid], dstv)
    plsc.subcore_barrier()

    def body(j, carry):
        pltpu.async_copy(y_hbm.at[srcv.at[j]], rows_v, sem).wait()
        pltpu.sync_copy(rows_v, shared.at[dstv.at[j]], add=True)
        return carry

    lax.fori_loop(0, TJ, body, 0)
    plsc.subcore_barrier()
    pltpu.sync_copy(shared.at[pl.ds(s * ZROWS, ZROWS)],
                    out_hbm.at[c, pl.ds(s * ZROWS, ZROWS)])


@functools.partial(
    pl.kernel, mesh=_sc_mesh,
    out_type=jax.ShapeDtypeStruct((PPADLEN, 4 * D), jnp.float32),
    scratch_types=[
        pltpu.VMEM((PJ, CH), jnp.int32),
        pltpu.VMEM((PJ, CH), jnp.int32),
        pltpu.VMEM((CH, D), jnp.float32),
        pltpu.VMEM((CH, D), jnp.float32),
        pltpu.VMEM((CH, D), jnp.float32),
        pltpu.VMEM((CH, D), jnp.float32),
        pltpu.SemaphoreType.DMA,
    ],
)
def _pair_gather_kernel(h_hbm, x_hbm, u_hbm, v_hbm, out_hbm,
                        uv, vv, bhu, bhv, bxu, bxv, sem):
    c = lax.axis_index("c")
    s = lax.axis_index("s")
    wid = s * NC + c
    pltpu.sync_copy(u_hbm.at[wid], uv)
    pltpu.sync_copy(v_hbm.at[wid], vv)
    base = wid * PJ * CH

    def body(j, carry):
        pltpu.async_copy(h_hbm.at[uv.at[j]], bhu, sem).wait()
        pltpu.async_copy(h_hbm.at[vv.at[j]], bhv, sem).wait()
        pltpu.async_copy(x_hbm.at[uv.at[j]], bxu, sem).wait()
        pltpu.async_copy(x_hbm.at[vv.at[j]], bxv, sem).wait()
        r = base + j * CH
        pltpu.sync_copy(bhu, out_hbm.at[pl.ds(r, CH), pl.ds(0, D)])
        pltpu.sync_copy(bhv, out_hbm.at[pl.ds(r, CH), pl.ds(D, D)])
        pltpu.sync_copy(bxu, out_hbm.at[pl.ds(r, CH), pl.ds(2 * D, D)])
        pltpu.sync_copy(bxv, out_hbm.at[pl.ds(r, CH), pl.ds(3 * D, D)])
        return carry

    lax.fori_loop(0, PJ, body, 0)


def _enc1_body(x_ref, w_ref, degp_ref, y_ref, dinv_ref):
    deg = degp_ref[0, :N, 0:1] + degp_ref[1, :N, 0:1] + 1.0
    dinv = lax.rsqrt(deg)
    xw = jnp.dot(x_ref[...], w_ref[...], preferred_element_type=jnp.float32)
    y_ref[...] = xw * dinv
    dinv_ref[...] = jnp.broadcast_to(dinv, (N, H))


def _enc2_body(aggp_ref, y_ref, dinv_ref, b_ref, g_ref, be_ref, w2_ref,
               y2_ref):
    dinv = dinv_ref[...]
    h = dinv * (aggp_ref[0, :N] + aggp_ref[1, :N] + y_ref[...]) + b_ref[...]
    mean = jnp.mean(h, axis=0, keepdims=True)
    var = jnp.mean((h - mean) ** 2, axis=0, keepdims=True)
    h = (h - mean) * lax.rsqrt(var + EPS) * g_ref[...] + be_ref[...]
    h = jnp.maximum(h, 0.0)
    xw2 = jnp.dot(h, w2_ref[...], preferred_element_type=jnp.float32)
    y2_ref[...] = xw2 * dinv


def _enc3_body(aggp_ref, y_ref, dinv_ref, b_ref, g_ref, be_ref, h2_ref):
    dinv = dinv_ref[...]
    h = dinv * (aggp_ref[0, :N] + aggp_ref[1, :N] + y_ref[...]) + b_ref[...]
    mean = jnp.mean(h, axis=0, keepdims=True)
    var = jnp.mean((h - mean) ** 2, axis=0, keepdims=True)
    h2_ref[...] = (h - mean) * lax.rsqrt(var + EPS) * g_ref[...] + be_ref[...]


DEC_BLK = 512


def _dec_body(g_ref, mw1_ref, mb1_ref, mw2_ref, mb2_ref, mw3_ref, mb3_ref,
              out_ref):
    g = g_ref[...]
    hu = g[:, 0:D]
    hv = g[:, D:2 * D]
    xu = g[:, 2 * D:3 * D]
    xv = g[:, 3 * D:4 * D]
    mw1 = mw1_ref[...]
    z = (jnp.dot(jnp.abs(hu - hv), mw1[0:D],
                 preferred_element_type=jnp.float32)
         + jnp.dot(hu * hv, mw1[D:2 * D],
                   preferred_element_type=jnp.float32)
         + jnp.dot(jnp.abs(xu - xv), mw1[2 * D:3 * D],
                   preferred_element_type=jnp.float32)
         + jnp.dot(xu * xv, mw1[3 * D:4 * D],
                   preferred_element_type=jnp.float32)
         + mb1_ref[...])
    z = jnp.maximum(z, 0.0)
    z = jnp.dot(z, mw2_ref[...], preferred_element_type=jnp.float32)
    z = jnp.maximum(z + mb2_ref[...], 0.0)
    out_ref[...] = (jnp.dot(z, mw3_ref[...],
                            preferred_element_type=jnp.float32)
                    + mb3_ref[...])


def kernel(x, edge_index, edge_pairs, W1, b1, g1, be1, W2, b2, g2, be2,
           mW1, mb1, mW2, mb2, mW3, mb3):
    f32 = jnp.float32
    src = edge_index[0]
    dst = edge_index[1]
    epad = EPADLEN - E
    src3 = jnp.concatenate([src, jnp.zeros((epad,), jnp.int32)]
                           ).reshape(NW, TJ, CH)
    dst3 = jnp.concatenate([dst, jnp.full((epad,), N, jnp.int32)]
                           ).reshape(NW, TJ, CH)
    ppad = PPADLEN - P
    u3 = jnp.concatenate([edge_pairs[0], jnp.zeros((ppad,), jnp.int32)]
                         ).reshape(NW, PJ, CH)
    v3 = jnp.concatenate([edge_pairs[1], jnp.zeros((ppad,), jnp.int32)]
                         ).reshape(NW, PJ, CH)

    zeros16 = jnp.zeros((NPAD, 16), f32)
    ones16 = jnp.ones((CH, 16), f32)
    zerosD = jnp.zeros((NPAD, D), f32)

    degp = _deg_kernel(dst3, zeros16, ones16)

    y1, dinv2d = pl.pallas_call(
        _enc1_body,
        out_shape=(jax.ShapeDtypeStruct((N, H), f32),
                   jax.ShapeDtypeStruct((N, H), f32)),
    )(x, W1, degp)

    agg1p = _agg_kernel(y1, src3, dst3, zerosD)

    y2 = pl.pallas_call(
        _enc2_body,
        out_shape=jax.ShapeDtypeStruct((N, H), f32),
    )(agg1p, y1, dinv2d, b1.reshape(1, H), g1.reshape(1, H),
      be1.reshape(1, H), W2)

    agg2p = _agg_kernel(y2, src3, dst3, zerosD)

    h2 = pl.pallas_call(
        _enc3_body,
        out_shape=jax.ShapeDtypeStruct((N, H), f32),
    )(agg2p, y2, dinv2d, b2.reshape(1, H), g2.reshape(1, H),
      be2.reshape(1, H))

    gathered = _pair_gather_kernel(h2, x, u3, v3)

    nblk = PPADLEN // DEC_BLK
    logits = pl.pallas_call(
        _dec_body,
        grid=(nblk,),
        in_specs=[
            pl.BlockSpec((DEC_BLK, 4 * D), lambda i: (i, 0)),
            pl.BlockSpec((4 * D, H), lambda i: (0, 0)),
            pl.BlockSpec((1, H), lambda i: (0, 0)),
            pl.BlockSpec((H, H // 2), lambda i: (0, 0)),
            pl.BlockSpec((1, H // 2), lambda i: (0, 0)),
            pl.BlockSpec((H // 2, 1), lambda i: (0, 0)),
            pl.BlockSpec((1, 1), lambda i: (0, 0)),
        ],
        out_specs=pl.BlockSpec((DEC_BLK, 1), lambda i: (i, 0)),
        out_shape=jax.ShapeDtypeStruct((PPADLEN, 1), f32),
    )(gathered, mW1, mb1.reshape(1, H), mW2, mb2.reshape(1, H // 2),
      mW3, mb3.reshape(1, 1))

    return logits[:P, 0]


# speedup vs baseline: 9.2237x; 1.5484x over previous
import functools

import jax
import jax.numpy as jnp
from jax import lax
from jax.experimental import pallas as pl
from jax.experimental.pallas import tpu as pltpu
from jax.experimental.pallas import tpu_sc as plsc

N = 10000
E = 320000
P = 200000
D = 128
H = 128
EPS = 1e-5

NC = 2
NS = 16
NW = NC * NS
CH = 128
CA = 64
CP = 64

NPAD = 10240
TJ = 80
EPADLEN = NW * CH * TJ
TA = EPADLEN // (NW * CA)
BCH = 16
NBLK = TA // BCH
PJ = -(-P // (NW * CP))
PPADLEN = NW * CP * PJ
ZROWS = NPAD // NS

_sc_mesh = plsc.VectorSubcoreMesh(core_axis_name="c", subcore_axis_name="s")


@functools.partial(
    pl.kernel, mesh=_sc_mesh,
    out_type=jax.ShapeDtypeStruct((NC, NPAD, 16), jnp.float32),
    scratch_types=[
        pltpu.VMEM((TJ, CH), jnp.int32),
        pltpu.VMEM((CH, 16), jnp.float32),
        pltpu.VMEM_SHARED((NPAD, 16), jnp.float32),
    ],
)
def _deg_kernel(dst_hbm, zeros_hbm, ones_hbm, out_hbm, idx_v, ones_v, shared):
    c = lax.axis_index("c")
    s = lax.axis_index("s")
    wid = s * NC + c
    pltpu.sync_copy(zeros_hbm.at[pl.ds(s * ZROWS, ZROWS)],
                    shared.at[pl.ds(s * ZROWS, ZROWS)])
    pltpu.sync_copy(ones_hbm, ones_v)
    pltpu.sync_copy(dst_hbm.at[wid], idx_v)
    plsc.subcore_barrier()

    def body(j, carry):
        pltpu.sync_copy(ones_v, shared.at[idx_v.at[j]], add=True)
        return carry

    lax.fori_loop(0, TJ, body, 0)
    plsc.subcore_barrier()
    pltpu.sync_copy(shared.at[pl.ds(s * ZROWS, ZROWS)],
                    out_hbm.at[c, pl.ds(s * ZROWS, ZROWS)])


@functools.partial(
    pl.kernel, mesh=_sc_mesh,
    out_type=jax.ShapeDtypeStruct((NC, NPAD, D), jnp.float32),
    scratch_types=[
        pltpu.VMEM((BCH, CA), jnp.int32),
        pltpu.VMEM((BCH, CA), jnp.int32),
        pltpu.VMEM((CA, D), jnp.float32),
        pltpu.VMEM((CA, D), jnp.float32),
        pltpu.VMEM((CA, D), jnp.float32),
        pltpu.VMEM((CA, D), jnp.float32),
        pltpu.VMEM_SHARED((NPAD, D), jnp.float32),
        pltpu.SemaphoreType.DMA,
        pltpu.SemaphoreType.DMA,
        pltpu.SemaphoreType.DMA,
        pltpu.SemaphoreType.DMA,
        pltpu.SemaphoreType.DMA,
        pltpu.SemaphoreType.DMA,
        pltpu.SemaphoreType.DMA,
        pltpu.SemaphoreType.DMA,
    ],
)
def _agg_kernel(y_hbm, src_hbm, dst_hbm, zeros_hbm, out_hbm,
                srcv, dstv, b0, b1, b2, b3, shared,
                g0, g1, g2, g3, s0, s1, s2, s3):
    c = lax.axis_index("c")
    s = lax.axis_index("s")
    wid = s * NC + c
    pltpu.sync_copy(zeros_hbm.at[pl.ds(s * ZROWS, ZROWS)],
                    shared.at[pl.ds(s * ZROWS, ZROWS)])
    plsc.subcore_barrier()
    bufs = (b0, b1, b2, b3)
    gsems = (g0, g1, g2, g3)
    ssems = (s0, s1, s2, s3)

    ssem = ssems[0]

    for bi in range(NBLK):
        pltpu.sync_copy(src_hbm.at[wid, pl.ds(bi * BCH, BCH)], srcv)
        pltpu.sync_copy(dst_hbm.at[wid, pl.ds(bi * BCH, BCH)], dstv)
        for g in range(BCH // 4):
            for k in range(4):
                pltpu.async_copy(y_hbm.at[srcv.at[g * 4 + k]],
                                 bufs[k], gsems[k])
            for k in range(4):
                pltpu.make_async_copy(y_hbm.at[srcv.at[g * 4 + k]],
                                      bufs[k], gsems[k]).wait()
            for k in range(4):
                pltpu.sync_copy(bufs[k], shared.at[dstv.at[g * 4 + k]],
                                add=True)
    plsc.subcore_barrier()
    pltpu.sync_copy(shared.at[pl.ds(s * ZROWS, ZROWS)],
                    out_hbm.at[c, pl.ds(s * ZROWS, ZROWS)])


@functools.partial(
    pl.kernel, mesh=_sc_mesh,
    out_type=jax.ShapeDtypeStruct((EPADLEN, D), jnp.float32),
    scratch_types=[
        pltpu.VMEM((TA, CA), jnp.int32),
        pltpu.VMEM((CA, D), jnp.float32),
    ],
)
def _gdbg_kernel(y_hbm, src_hbm, out_hbm, srcv, buf):
    c = lax.axis_index("c")
    s = lax.axis_index("s")
    wid = s * NC + c
    pltpu.sync_copy(src_hbm.at[wid], srcv)
    base = wid * TA * CA

    def body(j, carry):
        pltpu.sync_copy(y_hbm.at[srcv.at[j]], buf)
        pltpu.sync_copy(buf, out_hbm.at[pl.ds(base + j * CA, CA)])
        return carry

    lax.fori_loop(0, TA, body, 0)


@functools.partial(
    pl.kernel, mesh=_sc_mesh,
    out_type=jax.ShapeDtypeStruct((PPADLEN, 4 * D), jnp.float32),
    scratch_types=[
        pltpu.VMEM((PJ, CP), jnp.int32),
        pltpu.VMEM((PJ, CP), jnp.int32),
        pltpu.VMEM((CP, 2 * D), jnp.float32),
        pltpu.VMEM((CP, 2 * D), jnp.float32),
        pltpu.VMEM((CP, 2 * D), jnp.float32),
        pltpu.VMEM((CP, 2 * D), jnp.float32),
        pltpu.SemaphoreType.DMA,
        pltpu.SemaphoreType.DMA,
    ],
)
def _pair_gather_kernel(t_hbm, u_hbm, v_hbm, out_hbm,
                        uv, vv, bu0, bv0, bu1, bv1, sem0, sem1):
    c = lax.axis_index("c")
    s = lax.axis_index("s")
    wid = s * NC + c
    pltpu.sync_copy(u_hbm.at[wid], uv)
    pltpu.sync_copy(v_hbm.at[wid], vv)
    base = wid * PJ * CP

    pltpu.async_copy(t_hbm.at[uv.at[0]], bu0, sem0)
    pltpu.async_copy(t_hbm.at[vv.at[0]], bv0, sem0)
    npair = PJ // 2

    def body(jj, carry):
        j0 = 2 * jj
        r0 = base + j0 * CP
        pltpu.make_async_copy(t_hbm.at[uv.at[j0]], bu0, sem0).wait()
        pltpu.make_async_copy(t_hbm.at[vv.at[j0]], bv0, sem0).wait()
        pltpu.async_copy(t_hbm.at[uv.at[j0 + 1]], bu1, sem1)
        pltpu.async_copy(t_hbm.at[vv.at[j0 + 1]], bv1, sem1)
        pltpu.sync_copy(bu0, out_hbm.at[pl.ds(r0, CP), pl.ds(0, 2 * D)])
        pltpu.sync_copy(bv0, out_hbm.at[pl.ds(r0, CP), pl.ds(2 * D, 2 * D)])
        pltpu.make_async_copy(t_hbm.at[uv.at[j0 + 1]], bu1, sem1).wait()
        pltpu.make_async_copy(t_hbm.at[vv.at[j0 + 1]], bv1, sem1).wait()

        @pl.when(jj + 1 < npair)
        def _():
            pltpu.async_copy(t_hbm.at[uv.at[j0 + 2]], bu0, sem0)
            pltpu.async_copy(t_hbm.at[vv.at[j0 + 2]], bv0, sem0)

        r1 = r0 + CP
        pltpu.sync_copy(bu1, out_hbm.at[pl.ds(r1, CP), pl.ds(0, 2 * D)])
        pltpu.sync_copy(bv1, out_hbm.at[pl.ds(r1, CP), pl.ds(2 * D, 2 * D)])
        return carry

    lax.fori_loop(0, npair, body, 0)


def _enc1_body(x_ref, w_ref, degp_ref, y_ref, dinv_ref):
    deg = degp_ref[0, :N, 0:1] + degp_ref[1, :N, 0:1] + 1.0
    dinv = lax.rsqrt(deg)
    xw = jnp.dot(x_ref[...], w_ref[...], preferred_element_type=jnp.float32)
    y_ref[...] = xw * dinv
    dinv_ref[...] = jnp.broadcast_to(dinv, (N, H))


def _enc2_body(aggp_ref, y_ref, dinv_ref, b_ref, g_ref, be_ref, w2_ref,
               y2_ref):
    dinv = dinv_ref[...]
    h = dinv * (aggp_ref[0, :N] + aggp_ref[1, :N] + y_ref[...]) + b_ref[...]
    mean = jnp.mean(h, axis=0, keepdims=True)
    var = jnp.mean((h - mean) ** 2, axis=0, keepdims=True)
    h = (h - mean) * lax.rsqrt(var + EPS) * g_ref[...] + be_ref[...]
    h = jnp.maximum(h, 0.0)
    xw2 = jnp.dot(h, w2_ref[...], preferred_element_type=jnp.float32)
    y2_ref[...] = xw2 * dinv


def _enc3_body(aggp_ref, y_ref, dinv_ref, b_ref, g_ref, be_ref, x_ref,
               t_ref):
    dinv = dinv_ref[...]
    h = dinv * (aggp_ref[0, :N] + aggp_ref[1, :N] + y_ref[...]) + b_ref[...]
    mean = jnp.mean(h, axis=0, keepdims=True)
    var = jnp.mean((h - mean) ** 2, axis=0, keepdims=True)
    t_ref[:, 0:H] = (h - mean) * lax.rsqrt(var + EPS) * g_ref[...] + be_ref[...]
    t_ref[:, H:H + D] = x_ref[...]


DEC_BLK = 512


def _dec_body(g_ref, mw1_ref, mb1_ref, mw2_ref, mb2_ref, mw3_ref, mb3_ref,
              out_ref):
    g = g_ref[...]
    hu = g[:, 0:D]
    xu = g[:, D:2 * D]
    hv = g[:, 2 * D:3 * D]
    xv = g[:, 3 * D:4 * D]
    mw1 = mw1_ref[...]
    z = (jnp.dot(jnp.abs(hu - hv), mw1[0:D],
                 preferred_element_type=jnp.float32)
         + jnp.dot(hu * hv, mw1[D:2 * D],
                   preferred_element_type=jnp.float32)
         + jnp.dot(jnp.abs(xu - xv), mw1[2 * D:3 * D],
                   preferred_element_type=jnp.float32)
         + jnp.dot(xu * xv, mw1[3 * D:4 * D],
                   preferred_element_type=jnp.float32)
         + mb1_ref[...])
    z = jnp.maximum(z, 0.0)
    z = jnp.dot(z, mw2_ref[...], preferred_element_type=jnp.float32)
    z = jnp.maximum(z + mb2_ref[...], 0.0)
    out_ref[...] = (jnp.dot(z, mw3_ref[...],
                            preferred_element_type=jnp.float32)
                    + mb3_ref[...])


def kernel(x, edge_index, edge_pairs, W1, b1, g1, be1, W2, b2, g2, be2,
           mW1, mb1, mW2, mb2, mW3, mb3):
    f32 = jnp.float32
    src = edge_index[0]
    dst = edge_index[1]
    epad = EPADLEN - E
    eiota = jnp.arange(epad, dtype=jnp.int32)
    srcp = jnp.concatenate([src, eiota % N])
    dstp = jnp.concatenate([dst, N + (eiota % (NPAD - N))])
    src3a = srcp.reshape(NW, TA, CA)
    dst3a = dstp.reshape(NW, TA, CA)
    dst3d = dstp.reshape(NW, TJ, CH)
    ppad = PPADLEN - P
    piota = jnp.arange(ppad, dtype=jnp.int32)
    u3 = jnp.concatenate([edge_pairs[0], piota % N]).reshape(NW, PJ, CP)
    v3 = jnp.concatenate([edge_pairs[1], (piota * 7) % N]).reshape(NW, PJ, CP)

    zeros16 = jnp.zeros((NPAD, 16), f32)
    ones16 = jnp.ones((CH, 16), f32)
    zerosD = jnp.zeros((NPAD, D), f32)

    degp = _deg_kernel(dst3d, zeros16, ones16)

    y1, dinv2d = pl.pallas_call(
        _enc1_body,
        out_shape=(jax.ShapeDtypeStruct((N, H), f32),
                   jax.ShapeDtypeStruct((N, H), f32)),
    )(x, W1, degp)

    agg1p = _agg_kernel(y1, src3a, dst3a, zerosD)

    y2 = pl.pallas_call(
        _enc2_body,
        out_shape=jax.ShapeDtypeStruct((N, H), f32),
    )(agg1p, y1, dinv2d, b1.reshape(1, H), g1.reshape(1, H),
      be1.reshape(1, H), W2)

    agg2p = _agg_kernel(y2, src3a, dst3a, zerosD)

    t = pl.pallas_call(
        _enc3_body,
        out_shape=jax.ShapeDtypeStruct((N, H + D), f32),
    )(agg2p, y2, dinv2d, b2.reshape(1, H), g2.reshape(1, H),
      be2.reshape(1, H), x)

    gathered = _pair_gather_kernel(t, u3, v3)

    nblk = PPADLEN // DEC_BLK
    logits = pl.pallas_call(
        _dec_body,
        grid=(nblk,),
        in_specs=[
            pl.BlockSpec((DEC_BLK, 4 * D), lambda i: (i, 0)),
            pl.BlockSpec((4 * D, H), lambda i: (0, 0)),
            pl.BlockSpec((1, H), lambda i: (0, 0)),
            pl.BlockSpec((H, H // 2), lambda i: (0, 0)),
            pl.BlockSpec((1, H // 2), lambda i: (0, 0)),
            pl.BlockSpec((H // 2, 1), lambda i: (0, 0)),
            pl.BlockSpec((1, 1), lambda i: (0, 0)),
        ],
        out_specs=pl.BlockSpec((DEC_BLK, 1), lambda i: (i, 0)),
        out_shape=jax.ShapeDtypeStruct((PPADLEN, 1), f32),
    )(gathered, mW1, mb1.reshape(1, H), mW2,
      mb2.reshape(1, H // 2), mW3, mb3.reshape(1, 1))

    return logits[:P, 0]
